# baseline (speedup 1.0000x reference)
import functools

import jax
import jax.numpy as jnp
from jax import lax
from jax.experimental import pallas as pl
from jax.experimental.pallas import tpu as pltpu
from jax.experimental.pallas import tpu_sc as plsc

NUM_SAMPLE = 32
_NW = 32
_C = 8
_ROWS = _C * NUM_SAMPLE
_LG = 8


def _sc_body(nb_hbm, out_hbm, nbuf, sums, sem, *, num_chunks):
    wid = lax.axis_index("s") * 2 + lax.axis_index("c")

    def start(i):
        c = wid + i * _NW
        slot = lax.rem(i, 2)

        @pl.when(c < num_chunks)
        def _():
            pltpu.async_copy(
                nb_hbm.at[pl.ds(c * _ROWS, _ROWS)], nbuf.at[slot], sem.at[slot]
            )

    start(jnp.int32(0))
    max_iters = (num_chunks + _NW - 1) // _NW

    def iter_body(i, carry):
        c = wid + i * _NW
        slot = lax.rem(i, 2)
        start(i + 1)

        @pl.when(c < num_chunks)
        def _():
            pltpu.make_async_copy(
                nb_hbm.at[pl.ds(c * _ROWS, _ROWS)], nbuf.at[slot], sem.at[slot]
            ).wait()

            def node_body(j, _):
                def rbody(r, acc):
                    row = j * NUM_SAMPLE + r
                    return tuple(
                        acc[k] + nbuf[slot, row, pl.ds(k * 16, 16)]
                        for k in range(_LG)
                    )

                acc = lax.fori_loop(
                    0, NUM_SAMPLE, rbody,
                    tuple(jnp.zeros((16,), jnp.float32) for _ in range(_LG)),
                )
                for k in range(_LG):
                    sums[j, pl.ds(k * 16, 16)] = acc[k]
                return 0

            lax.fori_loop(0, _C, node_body, 0)
            pltpu.sync_copy(sums, out_hbm.at[pl.ds(c * _C, _C)])
        return 0

    lax.fori_loop(0, max_iters, iter_body, 0)


def _sc_segment_sums(neighbor_feats):
    n_nodes = neighbor_feats.shape[0] // NUM_SAMPLE
    d = neighbor_feats.shape[1]
    num_chunks = n_nodes // _C
    mesh = plsc.VectorSubcoreMesh(core_axis_name="c", subcore_axis_name="s")
    body = functools.partial(_sc_body, num_chunks=num_chunks)
    return pl.kernel(
        body,
        out_type=jax.ShapeDtypeStruct((n_nodes, d), jnp.float32),
        mesh=mesh,
        scratch_types=[
            pltpu.VMEM((2, _ROWS, d), jnp.float32),
            pltpu.VMEM((_C, d), jnp.float32),
            pltpu.SemaphoreType.DMA((2,)),
        ],
    )(neighbor_feats)


def _tc_body(nf_ref, sums_ref, w1_ref, w2_ref, out_ref):
    top = jax.lax.dot_general(w1_ref[...], nf_ref[...], (((1,), (1,)), ((), ())),
                              preferred_element_type=jnp.float32)
    w2s = w2_ref[...] * (1.0 / NUM_SAMPLE)
    bot = jax.lax.dot_general(w2s, sums_ref[...], (((1,), (1,)), ((), ())),
                              preferred_element_type=jnp.float32)
    out_ref[...] = jnp.maximum(jnp.concatenate([top, bot], axis=0), 0.0)


def kernel(node_feats, neighbor_feats, node_count, W1, W2):
    n, d = node_feats.shape
    e2 = W1.shape[0]
    sums = _sc_segment_sums(neighbor_feats)
    BN = 512
    out = pl.pallas_call(
        _tc_body,
        grid=(pl.cdiv(n, BN),),
        in_specs=[
            pl.BlockSpec((BN, d), lambda i: (i, 0)),
            pl.BlockSpec((BN, d), lambda i: (i, 0)),
            pl.BlockSpec((e2, d), lambda i: (0, 0)),
            pl.BlockSpec((e2, d), lambda i: (0, 0)),
        ],
        out_specs=pl.BlockSpec((2 * e2, BN), lambda i: (0, i)),
        out_shape=jax.ShapeDtypeStruct((2 * e2, n), jnp.float32),
        compiler_params=pltpu.CompilerParams(
            dimension_semantics=("parallel",),
        ),
    )(node_feats, sums, W1, W2)
    return out


# speedup vs baseline: 1.3293x; 1.3293x over previous
import functools

import jax
import jax.numpy as jnp
from jax import lax
from jax.experimental import pallas as pl
from jax.experimental.pallas import tpu as pltpu
from jax.experimental.pallas import tpu_sc as plsc

NUM_SAMPLE = 32
_NW = 32
_C = 8
_ROWS = _C * NUM_SAMPLE
_LG = 8
_BN = 512
_NB1 = 11


def _sc_body(nb_hbm, out_hbm, nbuf, sums, sem, *, num_chunks, row0):
    wid = lax.axis_index("s") * 2 + lax.axis_index("c")

    def start(i):
        c = wid + i * _NW
        slot = lax.rem(i, 2)

        @pl.when(c < num_chunks)
        def _():
            pltpu.async_copy(
                nb_hbm.at[pl.ds(row0 + c * _ROWS, _ROWS)], nbuf.at[slot],
                sem.at[slot],
            )

    start(jnp.int32(0))
    max_iters = (num_chunks + _NW - 1) // _NW

    def iter_body(i, carry):
        c = wid + i * _NW
        slot = lax.rem(i, 2)
        start(i + 1)

        @pl.when(c < num_chunks)
        def _():
            pltpu.make_async_copy(
                nb_hbm.at[pl.ds(row0 + c * _ROWS, _ROWS)], nbuf.at[slot],
                sem.at[slot],
            ).wait()

            def node_body(j, _):
                def rbody(r, acc):
                    row = j * NUM_SAMPLE + 2 * r
                    loaded = tuple(
                        acc[k]
                        + nbuf[slot, row, pl.ds(k * 16, 16)]
                        + nbuf[slot, row + 1, pl.ds(k * 16, 16)]
                        for k in range(_LG)
                    )
                    return loaded

                acc = lax.fori_loop(
                    0, NUM_SAMPLE // 2, rbody,
                    tuple(jnp.zeros((16,), jnp.float32) for _ in range(_LG)),
                )
                for k in range(_LG):
                    sums[j, pl.ds(k * 16, 16)] = acc[k]
                return 0

            lax.fori_loop(0, _C, node_body, 0)
            pltpu.sync_copy(sums, out_hbm.at[pl.ds(c * _C, _C)])
        return 0

    lax.fori_loop(0, max_iters, iter_body, 0)


def _sc_segment_sums(neighbor_feats, node0, n_nodes_sc):
    d = neighbor_feats.shape[1]
    num_chunks = pl.cdiv(n_nodes_sc, _C)
    mesh = plsc.VectorSubcoreMesh(core_axis_name="c", subcore_axis_name="s")
    body = functools.partial(
        _sc_body, num_chunks=num_chunks, row0=node0 * NUM_SAMPLE
    )
    return pl.kernel(
        body,
        out_type=jax.ShapeDtypeStruct((n_nodes_sc, d), jnp.float32),
        mesh=mesh,
        scratch_types=[
            pltpu.VMEM((2, _ROWS, d), jnp.float32),
            pltpu.VMEM((_C, d), jnp.float32),
            pltpu.SemaphoreType.DMA((2,)),
        ],
    )(neighbor_feats)


def _tc_fused_body(nf_ref, nb_ref, w1_ref, w2_ref, out_ref):
    bn = nf_ref.shape[0]
    d = nf_ref.shape[1]
    mean = jnp.mean(nb_ref[...].reshape(bn, NUM_SAMPLE, d), axis=1)
    top = jax.lax.dot_general(w1_ref[...], nf_ref[...], (((1,), (1,)), ((), ())),
                              preferred_element_type=jnp.float32)
    bot = jax.lax.dot_general(w2_ref[...], mean, (((1,), (1,)), ((), ())),
                              preferred_element_type=jnp.float32)
    out_ref[...] = jnp.maximum(jnp.concatenate([top, bot], axis=0), 0.0)


def _tc_tail_body(buf_ref, nf_ref, sums_ref, w1_ref, w2_ref, out_ref):
    del buf_ref
    top = jax.lax.dot_general(w1_ref[...], nf_ref[...], (((1,), (1,)), ((), ())),
                              preferred_element_type=jnp.float32)
    w2s = w2_ref[...] * (1.0 / NUM_SAMPLE)
    bot = jax.lax.dot_general(w2s, sums_ref[...], (((1,), (1,)), ((), ())),
                              preferred_element_type=jnp.float32)
    out_ref[...] = jnp.maximum(jnp.concatenate([top, bot], axis=0), 0.0)


def kernel(node_feats, neighbor_feats, node_count, W1, W2):
    n, d = node_feats.shape
    e2 = W1.shape[0]
    n1 = _NB1 * _BN
    n_sc = n - n1

    sums = _sc_segment_sums(neighbor_feats, n1, n_sc)

    out_a = pl.pallas_call(
        _tc_fused_body,
        grid=(_NB1,),
        in_specs=[
            pl.BlockSpec((_BN, d), lambda i: (i, 0)),
            pl.BlockSpec((_BN * NUM_SAMPLE, d), lambda i: (i, 0)),
            pl.BlockSpec((e2, d), lambda i: (0, 0)),
            pl.BlockSpec((e2, d), lambda i: (0, 0)),
        ],
        out_specs=pl.BlockSpec((2 * e2, _BN), lambda i: (0, i)),
        out_shape=jax.ShapeDtypeStruct((2 * e2, n), jnp.float32),
        compiler_params=pltpu.CompilerParams(
            dimension_semantics=("parallel",),
        ),
    )(node_feats, neighbor_feats, W1, W2)

    nb_tail = pl.cdiv(n_sc, _BN)
    out = pl.pallas_call(
        _tc_tail_body,
        grid=(nb_tail,),
        in_specs=[
            pl.BlockSpec(memory_space=pltpu.HBM),
            pl.BlockSpec((_BN, d), lambda i: (i + _NB1, 0)),
            pl.BlockSpec((_BN, d), lambda i: (i, 0)),
            pl.BlockSpec((e2, d), lambda i: (0, 0)),
            pl.BlockSpec((e2, d), lambda i: (0, 0)),
        ],
        out_specs=pl.BlockSpec((2 * e2, _BN), lambda i: (0, i + _NB1)),
        out_shape=jax.ShapeDtypeStruct((2 * e2, n), jnp.float32),
        input_output_aliases={0: 0},
        compiler_params=pltpu.CompilerParams(
            dimension_semantics=("parallel",),
        ),
    )(out_a, node_feats, sums, W1, W2)
    return out


# speedup vs baseline: 1.9321x; 1.4535x over previous
import jax
import jax.numpy as jnp
from jax.experimental import pallas as pl
from jax.experimental.pallas import tpu as pltpu

NUM_SAMPLE = 32


def _body(nf_ref, nb_ref, w1_ref, w2_ref, out_ref):
    bn = nf_ref.shape[0]
    d = nf_ref.shape[1]
    nb = nb_ref[...]
    mean = jnp.mean(nb.reshape(bn, NUM_SAMPLE, d), axis=1)
    nf = nf_ref[...]
    top = jax.lax.dot_general(w1_ref[...], nf, (((1,), (1,)), ((), ())),
                              preferred_element_type=jnp.float32)
    bot = jax.lax.dot_general(w2_ref[...], mean, (((1,), (1,)), ((), ())),
                              preferred_element_type=jnp.float32)
    out_ref[...] = jnp.maximum(jnp.concatenate([top, bot], axis=0), 0.0)


def kernel(node_feats, neighbor_feats, node_count, W1, W2):
    n, d = node_feats.shape
    e2 = W1.shape[0]
    BN = 640
    grid = (pl.cdiv(n, BN),)
    out = pl.pallas_call(
        _body,
        grid=grid,
        in_specs=[
            pl.BlockSpec((BN, d), lambda i: (i, 0)),
            pl.BlockSpec((BN * NUM_SAMPLE, d), lambda i: (i, 0)),
            pl.BlockSpec((e2, d), lambda i: (0, 0)),
            pl.BlockSpec((e2, d), lambda i: (0, 0)),
        ],
        out_specs=pl.BlockSpec((2 * e2, BN), lambda i: (0, i)),
        out_shape=jax.ShapeDtypeStruct((2 * e2, n), jnp.float32),
        compiler_params=pltpu.CompilerParams(
            dimension_semantics=("parallel",),
        ),
    )(node_feats, neighbor_feats, W1, W2)
    return out


# speedup vs baseline: 1.9373x; 1.0027x over previous
import jax
import jax.numpy as jnp
from jax.experimental import pallas as pl
from jax.experimental.pallas import tpu as pltpu

NUM_SAMPLE = 32


def _body(nf_ref, nba_ref, nbb_ref, w1_ref, w2_ref, out_ref):
    bn = nf_ref.shape[0]
    d = nf_ref.shape[1]
    half = bn // 2
    mean_a = jnp.mean(nba_ref[...].reshape(half, NUM_SAMPLE, d), axis=1)
    mean_b = jnp.mean(nbb_ref[...].reshape(half, NUM_SAMPLE, d), axis=1)
    mean = jnp.concatenate([mean_a, mean_b], axis=0)
    top = jax.lax.dot_general(w1_ref[...], nf_ref[...], (((1,), (1,)), ((), ())),
                              preferred_element_type=jnp.float32)
    bot = jax.lax.dot_general(w2_ref[...], mean, (((1,), (1,)), ((), ())),
                              preferred_element_type=jnp.float32)
    out_ref[...] = jnp.maximum(jnp.concatenate([top, bot], axis=0), 0.0)


def kernel(node_feats, neighbor_feats, node_count, W1, W2):
    n, d = node_feats.shape
    e2 = W1.shape[0]
    BN = 512
    half_rows = BN * NUM_SAMPLE // 2
    grid = (pl.cdiv(n, BN),)
    out = pl.pallas_call(
        _body,
        grid=grid,
        in_specs=[
            pl.BlockSpec((BN, d), lambda i: (i, 0)),
            pl.BlockSpec((half_rows, d), lambda i: (2 * i, 0)),
            pl.BlockSpec((half_rows, d), lambda i: (2 * i + 1, 0)),
            pl.BlockSpec((e2, d), lambda i: (0, 0)),
            pl.BlockSpec((e2, d), lambda i: (0, 0)),
        ],
        out_specs=pl.BlockSpec((2 * e2, BN), lambda i: (0, i)),
        out_shape=jax.ShapeDtypeStruct((2 * e2, n), jnp.float32),
        compiler_params=pltpu.CompilerParams(
            dimension_semantics=("parallel",),
        ),
    )(node_feats, neighbor_feats, neighbor_feats, W1, W2)
    return out


# speedup vs baseline: 1.9390x; 1.0009x over previous
import jax
import jax.numpy as jnp
from jax.experimental import pallas as pl
from jax.experimental.pallas import tpu as pltpu

NUM_SAMPLE = 32


def _body(nf_ref, nb_ref, w1_ref, w2_ref, out_ref):
    bn = nf_ref.shape[0]
    d = nf_ref.shape[1]
    nb = nb_ref[...]
    mean = jnp.mean(nb.reshape(bn, NUM_SAMPLE, d), axis=1)
    nf = nf_ref[...]
    top = jax.lax.dot_general(w1_ref[...], nf, (((1,), (1,)), ((), ())),
                              preferred_element_type=jnp.float32)
    bot = jax.lax.dot_general(w2_ref[...], mean, (((1,), (1,)), ((), ())),
                              preferred_element_type=jnp.float32)
    out_ref[...] = jnp.maximum(jnp.concatenate([top, bot], axis=0), 0.0)


def kernel(node_feats, neighbor_feats, node_count, W1, W2):
    n, d = node_feats.shape
    e2 = W1.shape[0]
    BN = 512
    grid = (pl.cdiv(n, BN),)
    out = pl.pallas_call(
        _body,
        grid=grid,
        in_specs=[
            pl.BlockSpec((BN, d), lambda i: (i, 0)),
            pl.BlockSpec((BN * NUM_SAMPLE, d), lambda i: (i, 0)),
            pl.BlockSpec((e2, d), lambda i: (0, 0)),
            pl.BlockSpec((e2, d), lambda i: (0, 0)),
        ],
        out_specs=pl.BlockSpec((2 * e2, BN), lambda i: (0, i)),
        out_shape=jax.ShapeDtypeStruct((2 * e2, n), jnp.float32),
        compiler_params=pltpu.CompilerParams(
            dimension_semantics=("parallel",),
        ),
    )(node_feats, neighbor_feats, W1, W2)
    return out
